# baseline (speedup 1.0000x reference)
import functools

import jax
import jax.numpy as jnp
from jax import lax
from jax.experimental import pallas as pl
from jax.experimental.pallas import tpu as pltpu
from jax.experimental.pallas import tpu_sc as plsc

_NW = 32
_B = 16384 * 32
_D = 128
_V = 32
_G = 128
_PER_W = _B // _NW
_NGRP = _PER_W // _G
_NB = 3
_NT = _NGRP // (2 * _NB)

_mesh = plsc.VectorSubcoreMesh(core_axis_name="c", subcore_axis_name="s")


@functools.partial(
    pl.kernel,
    mesh=_mesh,
    out_type=jax.ShapeDtypeStruct((_B, _D), jnp.float32),
    compiler_params=pltpu.CompilerParams(needs_layout_passes=False),
    scratch_types=[
        pltpu.VMEM((_NGRP, _G), jnp.int32),
        pltpu.VMEM((_G, _D), jnp.float32),
        pltpu.VMEM((_G, _D), jnp.float32),
        pltpu.VMEM((_G, _D), jnp.float32),
        pltpu.VMEM((_G, _D), jnp.float32),
        pltpu.VMEM((_G, _D), jnp.float32),
        pltpu.VMEM((_G, _D), jnp.float32),
        pltpu.VMEM_SHARED((_V, _D), jnp.float32),
        pltpu.SemaphoreType.DMA,
        pltpu.SemaphoreType.DMA,
        pltpu.SemaphoreType.DMA,
        pltpu.SemaphoreType.DMA,
        pltpu.SemaphoreType.DMA,
        pltpu.SemaphoreType.DMA,
        pltpu.SemaphoreType.DMA,
        pltpu.SemaphoreType.DMA,
        pltpu.SemaphoreType.DMA,
        pltpu.SemaphoreType.DMA,
        pltpu.SemaphoreType.DMA,
        pltpu.SemaphoreType.DMA,
    ],
)
def _gather_all(idx_hbm, table_hbm, out_hbm, idx_v,
                b0, b1, b2, b3, b4, b5, tab_sh,
                g0, g1, g2, g3, g4, g5, w0, w1, w2, w3, w4, w5):
    sid = lax.axis_index("s")
    wid = sid * 2 + lax.axis_index("c")
    base = wid * _PER_W

    @pl.when(sid == 0)
    def _():
        pltpu.sync_copy(table_hbm, tab_sh)

    pltpu.sync_copy(idx_hbm.at[wid], idx_v)
    plsc.subcore_barrier()

    bufs = (b0, b1, b2, b3, b4, b5)
    gsems = (g0, g1, g2, g3, g4, g5)
    wsems = (w0, w1, w2, w3, w4, w5)
    bank_a = (0, 1, 2)
    bank_b = (3, 4, 5)

    def g_start(b, g):
        pltpu.async_copy(tab_sh.at[idx_v.at[g]], bufs[b], gsems[b])

    def g_wait(b):
        pltpu.make_async_copy(tab_sh.at[idx_v.at[0]], bufs[b],
                              gsems[b]).wait()

    def w_start(b, g):
        pltpu.async_copy(bufs[b], out_hbm.at[pl.ds(base + g * _G, _G)],
                         wsems[b])

    def w_wait(b):
        pltpu.make_async_copy(bufs[b], out_hbm.at[pl.ds(base, _G)],
                              wsems[b]).wait()

    def drain_and_write(bank, ga):
        for i, b in enumerate(bank):
            g_wait(b)
            w_start(b, ga + i)

    def refill(bank, ga, first=False):
        for i, b in enumerate(bank):
            if not first:
                w_wait(b)
            g_start(b, jnp.minimum(ga + i, _NGRP - 1))

    refill(bank_a, 0, first=True)
    drain_and_write(bank_a, 0)
    refill(bank_b, _NB, first=True)
    drain_and_write(bank_b, _NB)
    refill(bank_a, 2 * _NB)

    def body(t, carry):
        ga = 2 * _NB * t
        drain_and_write(bank_a, ga)
        refill(bank_b, ga + _NB)
        drain_and_write(bank_b, ga + _NB)
        refill(bank_a, ga + 2 * _NB)
        return carry

    lax.fori_loop(1, _NT, body, 0)

    _TAIL = _NGRP - 2 * _NB * _NT
    for i, b in enumerate(bank_a):
        g_wait(b)
        if i < _TAIL:
            w_start(b, 2 * _NB * _NT + i)
    for b in bank_b:
        w_wait(b)
    for i, b in enumerate(bank_a):
        if i < _TAIL:
            w_wait(b)


def kernel(x, weight):
    xr = x.reshape(_NW, _NGRP, _G)
    out = _gather_all(xr, weight)
    return out.reshape(16384, 32, _D)
